# Initial kernel scaffold; baseline (speedup 1.0000x reference)
import jax
import jax.numpy as jnp
from jax.experimental import pallas as pl


def kernel(x_paper, x_author, ei_cites, ei_writes, ei_rev, Wl_c1, bl_c1, Wr_c1, Wl_w1, bl_w1, Wr_w1, Wl_r1, bl_r1, Wr_r1, Wl_c2, bl_c2, Wr_c2, Wl_w2, bl_w2, Wr_w2, Wl_r2, bl_r2, Wr_r2, W_lin, b_lin):
    raise NotImplementedError("write your pallas kernel here")


# speedup vs baseline: 2.0482x; 2.0482x over previous
import functools

import jax
import jax.numpy as jnp
from jax import lax
from jax.experimental import pallas as pl
from jax.experimental.pallas import tpu as pltpu
from jax.experimental.pallas import tpu_sc as plsc

N_PAPER = 10000
N_AUTHOR = 5000
D = 256
OUT = 64
E = 160000
CH = 128
NCH = 80
EP = 16 * NCH * CH
ACC_ROWS = 10112
ZR = 632


def _prep_edges(ei, n_dst):
    src = ei[0].astype(jnp.int32)
    dst = ei[1].astype(jnp.int32)
    pad = EP - E
    srcp = jnp.concatenate([src, jnp.zeros((pad,), jnp.int32)])
    dstp = jnp.concatenate([dst, jnp.full((pad,), n_dst, jnp.int32)])
    src4 = jnp.stack([2 * srcp, 2 * srcp + 1]).reshape(32, NCH, CH)
    dst3 = dstp.reshape(16, NCH, CH)
    return src4, dst3


def _sc_phase(table, src4, dst3, isrc, idst, stage, acc, sem, c, s):
    pltpu.sync_copy(src4.at[c * 16 + s], isrc)
    pltpu.sync_copy(dst3.at[s], idst)

    def body(j, carry):
        pltpu.async_copy(table.at[isrc.at[j]], stage, sem).wait()
        pltpu.sync_copy(stage, acc.at[idst.at[j]], add=True)
        return carry

    lax.fori_loop(0, NCH, body, 0)


def _dump_papers(acc, out, c, s):
    @pl.when(s < 14)
    def _():
        r0 = s * 624
        pltpu.sync_copy(acc.at[pl.ds(r0, 624)],
                        out.at[pl.ds(c * N_PAPER + r0, 624)])

    @pl.when(s >= 14)
    def _():
        r0 = 8736 + (s - 14) * 632
        pltpu.sync_copy(acc.at[pl.ds(r0, 632)],
                        out.at[pl.ds(c * N_PAPER + r0, 632)])


def _dump_authors(acc, out, c, s):
    @pl.when(s < 15)
    def _():
        r0 = s * 312
        pltpu.sync_copy(acc.at[pl.ds(r0, 312)],
                        out.at[pl.ds(c * N_AUTHOR + r0, 312)])

    @pl.when(s >= 15)
    def _():
        pltpu.sync_copy(acc.at[pl.ds(4680, 320)],
                        out.at[pl.ds(c * N_AUTHOR + 4680, 320)])


def _sc_counts(dc, dw, dr, zrows, ones128):
    mesh = plsc.VectorSubcoreMesh(core_axis_name="c", subcore_axis_name="s")
    f32 = jnp.float32
    out_type = [
        jax.ShapeDtypeStruct((N_PAPER, 128), f32),
        jax.ShapeDtypeStruct((N_PAPER, 128), f32),
        jax.ShapeDtypeStruct((N_AUTHOR, 128), f32),
    ]
    scratch = [
        pltpu.VMEM((NCH, CH), jnp.int32),
        pltpu.VMEM((CH, 128), f32),
        pltpu.VMEM_SHARED((ACC_ROWS, 128), f32),
        pltpu.SemaphoreType.DMA,
    ]

    @functools.partial(pl.kernel, mesh=mesh, out_type=out_type,
                       scratch_types=scratch)
    def k(dc_h, dw_h, dr_h, z_h, o_h, cc_o, cw_o, cr_o,
          idst, stage, acc, sem):
        c = lax.axis_index("c")
        s = lax.axis_index("s")
        pltpu.sync_copy(o_h, stage)

        def count_rel(d3, out, dump):
            pltpu.sync_copy(z_h, acc.at[pl.ds(s * ZR, ZR)])
            plsc.subcore_barrier()
            pltpu.sync_copy(d3.at[s], idst)

            def body(j, carry):
                pltpu.sync_copy(stage, acc.at[idst.at[j]], add=True)
                return carry

            lax.fori_loop(0, NCH, body, 0)
            plsc.subcore_barrier()
            dump(acc, out, 0, s)
            plsc.subcore_barrier()

        @pl.when(c == 0)
        def _():
            count_rel(dc_h, cc_o, _dump_papers)

        @pl.when(c == 1)
        def _():
            count_rel(dw_h, cw_o, _dump_papers)
            count_rel(dr_h, cr_o, _dump_authors)

    return k(dc, dw, dr, zrows, ones128)


def _sc_layer1(xpr, xar, sc_c, dc, sc_w, dw, sc_r, dr, zrows):
    mesh = plsc.VectorSubcoreMesh(core_axis_name="c", subcore_axis_name="s")
    f32 = jnp.float32
    out_type = [
        jax.ShapeDtypeStruct((2 * N_PAPER, 128), f32),
        jax.ShapeDtypeStruct((2 * N_PAPER, 128), f32),
        jax.ShapeDtypeStruct((2 * N_AUTHOR, 128), f32),
    ]
    scratch = [
        pltpu.VMEM((NCH, CH), jnp.int32),
        pltpu.VMEM((NCH, CH), jnp.int32),
        pltpu.VMEM((CH, 128), f32),
        pltpu.VMEM_SHARED((ACC_ROWS, 128), f32),
        pltpu.SemaphoreType.DMA,
    ]

    @functools.partial(pl.kernel, mesh=mesh, out_type=out_type,
                       scratch_types=scratch)
    def k(xpr_h, xar_h, sc_c_h, dc_h, sc_w_h, dw_h, sc_r_h, dr_h, z_h,
          ac_o, aw_o, ar_o, isrc, idst, stage, acc, sem):
        c = lax.axis_index("c")
        s = lax.axis_index("s")
        rels = [(xpr_h, sc_c_h, dc_h, ac_o, _dump_papers),
                (xar_h, sc_w_h, dw_h, aw_o, _dump_papers),
                (xpr_h, sc_r_h, dr_h, ar_o, _dump_authors)]
        for tab, s4, d3, a_o, dump in rels:
            pltpu.sync_copy(z_h, acc.at[pl.ds(s * ZR, ZR)])
            plsc.subcore_barrier()
            _sc_phase(tab, s4, d3, isrc, idst, stage, acc, sem, c, s)
            plsc.subcore_barrier()
            dump(acc, a_o, c, s)
            plsc.subcore_barrier()

    return k(xpr, xar, sc_c, dc, sc_w, dw, sc_r, dr, zrows)


def _sc_layer2(xpr, xar, sc_c, dc, sc_w, dw, zrows):
    mesh = plsc.VectorSubcoreMesh(core_axis_name="c", subcore_axis_name="s")
    f32 = jnp.float32
    out_type = [
        jax.ShapeDtypeStruct((2 * N_PAPER, 128), f32),
        jax.ShapeDtypeStruct((2 * N_PAPER, 128), f32),
    ]
    scratch = [
        pltpu.VMEM((NCH, CH), jnp.int32),
        pltpu.VMEM((NCH, CH), jnp.int32),
        pltpu.VMEM((CH, 128), f32),
        pltpu.VMEM_SHARED((ACC_ROWS, 128), f32),
        pltpu.SemaphoreType.DMA,
    ]

    @functools.partial(pl.kernel, mesh=mesh, out_type=out_type,
                       scratch_types=scratch)
    def k(xpr_h, xar_h, sc_c_h, dc_h, sc_w_h, dw_h, z_h,
          ac_o, aw_o, isrc, idst, stage, acc, sem):
        c = lax.axis_index("c")
        s = lax.axis_index("s")
        for tab, s4, d3, a_o in [(xpr_h, sc_c_h, dc_h, ac_o),
                                 (xar_h, sc_w_h, dw_h, aw_o)]:
            pltpu.sync_copy(z_h, acc.at[pl.ds(s * ZR, ZR)])
            plsc.subcore_barrier()
            _sc_phase(tab, s4, d3, isrc, idst, stage, acc, sem, c, s)
            plsc.subcore_barrier()
            _dump_papers(acc, a_o, c, s)
            plsc.subcore_barrier()

    return k(xpr, xar, sc_c, dc, sc_w, dw, zrows)


def _tc_paper(Ac, Aw, cc, cw, x, Wlc, Wlw, Wrc, Wrw, blc, blw):
    BM = 1000
    f32 = jnp.float32

    def body(ac_ref, aw_ref, cc_ref, cw_ref, x_ref, wlc, wlw, wrc, wrw,
             bc, bw, o_ref):
        inv_c = 1.0 / jnp.maximum(cc_ref[:, 0:1], 1.0)
        inv_w = 1.0 / jnp.maximum(cw_ref[:, 0:1], 1.0)
        p = (jnp.dot(ac_ref[0] * inv_c, wlc[0:128, :], preferred_element_type=f32)
             + jnp.dot(ac_ref[1] * inv_c, wlc[128:256, :], preferred_element_type=f32)
             + jnp.dot(aw_ref[0] * inv_w, wlw[0:128, :], preferred_element_type=f32)
             + jnp.dot(aw_ref[1] * inv_w, wlw[128:256, :], preferred_element_type=f32)
             + jnp.dot(x_ref[...], wrc[...] + wrw[...], preferred_element_type=f32)
             + bc[...] + bw[...])
        p = jnp.where(p >= 0, p, 0.01 * p)
        o_ref[...] = p.reshape(BM, 2, 128)

    grid = (N_PAPER // BM,)
    return pl.pallas_call(
        body,
        grid=grid,
        in_specs=[
            pl.BlockSpec((2, BM, 128), lambda i: (0, i, 0)),
            pl.BlockSpec((2, BM, 128), lambda i: (0, i, 0)),
            pl.BlockSpec((BM, 128), lambda i: (i, 0)),
            pl.BlockSpec((BM, 128), lambda i: (i, 0)),
            pl.BlockSpec((BM, D), lambda i: (i, 0)),
            pl.BlockSpec((D, D), lambda i: (0, 0)),
            pl.BlockSpec((D, D), lambda i: (0, 0)),
            pl.BlockSpec((D, D), lambda i: (0, 0)),
            pl.BlockSpec((D, D), lambda i: (0, 0)),
            pl.BlockSpec((1, D), lambda i: (0, 0)),
            pl.BlockSpec((1, D), lambda i: (0, 0)),
        ],
        out_specs=pl.BlockSpec((BM, 2, 128), lambda i: (i, 0, 0)),
        out_shape=jax.ShapeDtypeStruct((N_PAPER, 2, 128), f32),
    )(Ac, Aw, cc, cw, x, Wlc, Wlw, Wrc, Wrw, blc, blw)


def _tc_author(Ar, cr, x, Wlr, Wrr, blr):
    BM = 1000
    f32 = jnp.float32

    def body(ar_ref, cr_ref, x_ref, wlr, wrr, br, o_ref):
        inv_r = 1.0 / jnp.maximum(cr_ref[:, 0:1], 1.0)
        p = (jnp.dot(ar_ref[0] * inv_r, wlr[0:128, :], preferred_element_type=f32)
             + jnp.dot(ar_ref[1] * inv_r, wlr[128:256, :], preferred_element_type=f32)
             + jnp.dot(x_ref[...], wrr[...], preferred_element_type=f32)
             + br[...])
        p = jnp.where(p >= 0, p, 0.01 * p)
        o_ref[...] = p.reshape(BM, 2, 128)

    grid = (N_AUTHOR // BM,)
    return pl.pallas_call(
        body,
        grid=grid,
        in_specs=[
            pl.BlockSpec((2, BM, 128), lambda i: (0, i, 0)),
            pl.BlockSpec((BM, 128), lambda i: (i, 0)),
            pl.BlockSpec((BM, D), lambda i: (i, 0)),
            pl.BlockSpec((D, D), lambda i: (0, 0)),
            pl.BlockSpec((D, D), lambda i: (0, 0)),
            pl.BlockSpec((1, D), lambda i: (0, 0)),
        ],
        out_specs=pl.BlockSpec((BM, 2, 128), lambda i: (i, 0, 0)),
        out_shape=jax.ShapeDtypeStruct((N_AUTHOR, 2, 128), f32),
    )(Ar, cr, x, Wlr, Wrr, blr)


def _tc_final(Ac, Aw, cc, cw, xp, Wlc, Wlw, Wrc, Wrw, blc, blw, Wlin, blin):
    BM = 1000
    f32 = jnp.float32

    def body(ac_ref, aw_ref, cc_ref, cw_ref, x_ref, wlc, wlw, wrc, wrw,
             bc, bw, wl, bl, o_ref):
        inv_c = 1.0 / jnp.maximum(cc_ref[:, 0:1], 1.0)
        inv_w = 1.0 / jnp.maximum(cw_ref[:, 0:1], 1.0)
        x = x_ref[...].reshape(BM, D)
        p = (jnp.dot(ac_ref[0] * inv_c, wlc[0:128, :], preferred_element_type=f32)
             + jnp.dot(ac_ref[1] * inv_c, wlc[128:256, :], preferred_element_type=f32)
             + jnp.dot(aw_ref[0] * inv_w, wlw[0:128, :], preferred_element_type=f32)
             + jnp.dot(aw_ref[1] * inv_w, wlw[128:256, :], preferred_element_type=f32)
             + jnp.dot(x, wrc[...] + wrw[...], preferred_element_type=f32)
             + bc[...] + bw[...])
        p = jnp.where(p >= 0, p, 0.01 * p)
        o_ref[...] = jnp.dot(p, wl[...], preferred_element_type=f32) + bl[...]

    grid = (N_PAPER // BM,)
    return pl.pallas_call(
        body,
        grid=grid,
        in_specs=[
            pl.BlockSpec((2, BM, 128), lambda i: (0, i, 0)),
            pl.BlockSpec((2, BM, 128), lambda i: (0, i, 0)),
            pl.BlockSpec((BM, 128), lambda i: (i, 0)),
            pl.BlockSpec((BM, 128), lambda i: (i, 0)),
            pl.BlockSpec((BM, 2, 128), lambda i: (i, 0, 0)),
            pl.BlockSpec((D, D), lambda i: (0, 0)),
            pl.BlockSpec((D, D), lambda i: (0, 0)),
            pl.BlockSpec((D, D), lambda i: (0, 0)),
            pl.BlockSpec((D, D), lambda i: (0, 0)),
            pl.BlockSpec((1, D), lambda i: (0, 0)),
            pl.BlockSpec((1, D), lambda i: (0, 0)),
            pl.BlockSpec((D, OUT), lambda i: (0, 0)),
            pl.BlockSpec((1, OUT), lambda i: (0, 0)),
        ],
        out_specs=pl.BlockSpec((BM, OUT), lambda i: (i, 0)),
        out_shape=jax.ShapeDtypeStruct((N_PAPER, OUT), f32),
    )(Ac, Aw, cc, cw, xp, Wlc, Wlw, Wrc, Wrw, blc, blw, Wlin, blin)


def kernel(x_paper, x_author, ei_cites, ei_writes, ei_rev,
           Wl_c1, bl_c1, Wr_c1, Wl_w1, bl_w1, Wr_w1, Wl_r1, bl_r1, Wr_r1,
           Wl_c2, bl_c2, Wr_c2, Wl_w2, bl_w2, Wr_w2, Wl_r2, bl_r2, Wr_r2,
           W_lin, b_lin):
    f32 = jnp.float32
    sc_c, dc = _prep_edges(ei_cites, N_PAPER)
    sc_w, dw = _prep_edges(ei_writes, N_PAPER)
    sc_r, dr = _prep_edges(ei_rev, N_AUTHOR)
    zrows = jnp.zeros((ZR, 128), f32)
    ones128 = jnp.ones((CH, 128), f32)

    xpr = x_paper.reshape(2 * N_PAPER, 128)
    xar = x_author.reshape(2 * N_AUTHOR, 128)
    cc, cw, cr = _sc_counts(dc, dw, dr, zrows, ones128)
    Ac, Aw, Ar = _sc_layer1(xpr, xar, sc_c, dc, sc_w, dw, sc_r, dr, zrows)

    xp1 = _tc_paper(Ac.reshape(2, N_PAPER, 128), Aw.reshape(2, N_PAPER, 128),
                    cc, cw, x_paper, Wl_c1, Wl_w1, Wr_c1, Wr_w1,
                    bl_c1.reshape(1, D), bl_w1.reshape(1, D))
    xa1 = _tc_author(Ar.reshape(2, N_AUTHOR, 128), cr, x_author,
                     Wl_r1, Wr_r1, bl_r1.reshape(1, D))

    Ac2, Aw2 = _sc_layer2(xp1.reshape(2 * N_PAPER, 128),
                          xa1.reshape(2 * N_AUTHOR, 128),
                          sc_c, dc, sc_w, dw, zrows)

    return _tc_final(Ac2.reshape(2, N_PAPER, 128),
                     Aw2.reshape(2, N_PAPER, 128),
                     cc, cw, xp1, Wl_c2, Wl_w2, Wr_c2, Wr_w2,
                     bl_c2.reshape(1, D), bl_w2.reshape(1, D),
                     W_lin, b_lin.reshape(1, OUT))


# speedup vs baseline: 2.2741x; 1.1103x over previous
import functools

import jax
import jax.numpy as jnp
from jax import lax
from jax.experimental import pallas as pl
from jax.experimental.pallas import tpu as pltpu
from jax.experimental.pallas import tpu_sc as plsc

N_PAPER = 10000
N_AUTHOR = 5000
D = 256
OUT = 64
E = 160000
CH = 128
NCH = 80
IH = 40
EP = 16 * NCH * CH
ACC_ROWS = 10112
ZR = 632


def _prep_edges(ei, n_dst):
    src = ei[0].astype(jnp.int32)
    dst = ei[1].astype(jnp.int32)
    pad = EP - E
    srcp = jnp.concatenate([src, jnp.zeros((pad,), jnp.int32)])
    dstp = jnp.concatenate([dst, jnp.full((pad,), n_dst, jnp.int32)])
    src4 = jnp.stack([2 * srcp, 2 * srcp + 1]).reshape(64, IH, CH)
    dst3 = dstp.reshape(32, IH, CH)
    return src4, dst3


def _sc_phase(table, src4, dst3, isrc, idst, st0, st1, acc,
              gs0, gs1, ss0, ss1, c, s):
    w = c * 16 + s
    sts = (st0, st1)
    gss = (gs0, gs1)
    sss = (ss0, ss1)
    for h in range(2):
        pltpu.sync_copy(src4.at[w * 2 + h], isrc)
        pltpu.sync_copy(dst3.at[s * 2 + h], idst)
        pltpu.async_copy(table.at[isrc.at[0]], st0, gs0)
        pltpu.async_copy(table.at[isrc.at[1]], st1, gs1)

        def body(i, carry):
            for b in range(2):
                j = 2 * i + b
                pltpu.make_async_copy(table.at[isrc.at[j]], sts[b],
                                      gss[b]).wait()
                pltpu.async_copy(sts[b], acc.at[idst.at[j]], sss[b],
                                 add=True)
            for b in range(2):
                j = 2 * i + b
                pltpu.make_async_copy(sts[b], acc.at[idst.at[j]],
                                      sss[b]).wait()

                @pl.when(i < IH // 2 - 1)
                def _():
                    pltpu.async_copy(table.at[isrc.at[j + 2]], sts[b],
                                     gss[b])
            return carry

        lax.fori_loop(0, IH // 2, body, 0)


def _dump_papers(acc, out, c, s):
    @pl.when(s < 14)
    def _():
        r0 = s * 624
        pltpu.sync_copy(acc.at[pl.ds(r0, 624)],
                        out.at[pl.ds(c * N_PAPER + r0, 624)])

    @pl.when(s >= 14)
    def _():
        r0 = 8736 + (s - 14) * 632
        pltpu.sync_copy(acc.at[pl.ds(r0, 632)],
                        out.at[pl.ds(c * N_PAPER + r0, 632)])


def _dump_authors(acc, out, c, s):
    @pl.when(s < 15)
    def _():
        r0 = s * 312
        pltpu.sync_copy(acc.at[pl.ds(r0, 312)],
                        out.at[pl.ds(c * N_AUTHOR + r0, 312)])

    @pl.when(s >= 15)
    def _():
        pltpu.sync_copy(acc.at[pl.ds(4680, 320)],
                        out.at[pl.ds(c * N_AUTHOR + 4680, 320)])


def _sc_counts(dc, dw, dr, zrows, ones128):
    mesh = plsc.VectorSubcoreMesh(core_axis_name="c", subcore_axis_name="s")
    f32 = jnp.float32
    out_type = [
        jax.ShapeDtypeStruct((N_PAPER, 128), f32),
        jax.ShapeDtypeStruct((N_PAPER, 128), f32),
        jax.ShapeDtypeStruct((N_AUTHOR, 128), f32),
    ]
    scratch = [
        pltpu.VMEM((IH, CH), jnp.int32),
        pltpu.VMEM((CH, 128), f32),
        pltpu.VMEM_SHARED((ACC_ROWS, 128), f32),
        pltpu.SemaphoreType.DMA,
    ]

    @functools.partial(pl.kernel, mesh=mesh, out_type=out_type,
                       scratch_types=scratch)
    def k(dc_h, dw_h, dr_h, z_h, o_h, cc_o, cw_o, cr_o,
          idst, stage, acc, sem):
        c = lax.axis_index("c")
        s = lax.axis_index("s")
        pltpu.sync_copy(o_h, stage)

        def count_rel(d3, out, dump):
            pltpu.sync_copy(z_h, acc.at[pl.ds(s * ZR, ZR)])
            plsc.subcore_barrier()
            for h in range(2):
                pltpu.sync_copy(d3.at[s * 2 + h], idst)

                def body(i, carry):
                    for b in range(4):
                        pltpu.async_copy(stage, acc.at[idst.at[4 * i + b]],
                                         sem, add=True)
                    for b in range(4):
                        pltpu.make_async_copy(stage,
                                              acc.at[idst.at[4 * i + b]],
                                              sem).wait()
                    return carry

                lax.fori_loop(0, IH // 4, body, 0)
            plsc.subcore_barrier()
            dump(acc, out, 0, s)
            plsc.subcore_barrier()

        @pl.when(c == 0)
        def _():
            count_rel(dc_h, cc_o, _dump_papers)

        @pl.when(c == 1)
        def _():
            count_rel(dw_h, cw_o, _dump_papers)
            count_rel(dr_h, cr_o, _dump_authors)

    return k(dc, dw, dr, zrows, ones128)


def _sc_layer1(xpr, xar, sc_c, dc, sc_w, dw, sc_r, dr, zrows):
    mesh = plsc.VectorSubcoreMesh(core_axis_name="c", subcore_axis_name="s")
    f32 = jnp.float32
    out_type = [
        jax.ShapeDtypeStruct((2 * N_PAPER, 128), f32),
        jax.ShapeDtypeStruct((2 * N_PAPER, 128), f32),
        jax.ShapeDtypeStruct((2 * N_AUTHOR, 128), f32),
    ]
    scratch = [
        pltpu.VMEM((IH, CH), jnp.int32),
        pltpu.VMEM((IH, CH), jnp.int32),
        pltpu.VMEM((CH, 128), f32),
        pltpu.VMEM((CH, 128), f32),
        pltpu.VMEM_SHARED((ACC_ROWS, 128), f32),
        pltpu.SemaphoreType.DMA,
        pltpu.SemaphoreType.DMA,
        pltpu.SemaphoreType.DMA,
        pltpu.SemaphoreType.DMA,
    ]

    @functools.partial(pl.kernel, mesh=mesh, out_type=out_type,
                       scratch_types=scratch)
    def k(xpr_h, xar_h, sc_c_h, dc_h, sc_w_h, dw_h, sc_r_h, dr_h, z_h,
          ac_o, aw_o, ar_o, isrc, idst, st0, st1, acc, gs0, gs1, ss0, ss1):
        c = lax.axis_index("c")
        s = lax.axis_index("s")
        rels = [(xpr_h, sc_c_h, dc_h, ac_o, _dump_papers),
                (xar_h, sc_w_h, dw_h, aw_o, _dump_papers),
                (xpr_h, sc_r_h, dr_h, ar_o, _dump_authors)]
        for tab, s4, d3, a_o, dump in rels:
            pltpu.sync_copy(z_h, acc.at[pl.ds(s * ZR, ZR)])
            plsc.subcore_barrier()
            _sc_phase(tab, s4, d3, isrc, idst, st0, st1, acc,
                      gs0, gs1, ss0, ss1, c, s)
            plsc.subcore_barrier()
            dump(acc, a_o, c, s)
            plsc.subcore_barrier()

    return k(xpr, xar, sc_c, dc, sc_w, dw, sc_r, dr, zrows)


def _sc_layer2(xpr, xar, sc_c, dc, sc_w, dw, zrows):
    mesh = plsc.VectorSubcoreMesh(core_axis_name="c", subcore_axis_name="s")
    f32 = jnp.float32
    out_type = [
        jax.ShapeDtypeStruct((2 * N_PAPER, 128), f32),
        jax.ShapeDtypeStruct((2 * N_PAPER, 128), f32),
    ]
    scratch = [
        pltpu.VMEM((IH, CH), jnp.int32),
        pltpu.VMEM((IH, CH), jnp.int32),
        pltpu.VMEM((CH, 128), f32),
        pltpu.VMEM((CH, 128), f32),
        pltpu.VMEM_SHARED((ACC_ROWS, 128), f32),
        pltpu.SemaphoreType.DMA,
        pltpu.SemaphoreType.DMA,
        pltpu.SemaphoreType.DMA,
        pltpu.SemaphoreType.DMA,
    ]

    @functools.partial(pl.kernel, mesh=mesh, out_type=out_type,
                       scratch_types=scratch)
    def k(xpr_h, xar_h, sc_c_h, dc_h, sc_w_h, dw_h, z_h,
          ac_o, aw_o, isrc, idst, st0, st1, acc, gs0, gs1, ss0, ss1):
        c = lax.axis_index("c")
        s = lax.axis_index("s")
        for tab, s4, d3, a_o in [(xpr_h, sc_c_h, dc_h, ac_o),
                                 (xar_h, sc_w_h, dw_h, aw_o)]:
            pltpu.sync_copy(z_h, acc.at[pl.ds(s * ZR, ZR)])
            plsc.subcore_barrier()
            _sc_phase(tab, s4, d3, isrc, idst, st0, st1, acc,
                      gs0, gs1, ss0, ss1, c, s)
            plsc.subcore_barrier()
            _dump_papers(acc, a_o, c, s)
            plsc.subcore_barrier()

    return k(xpr, xar, sc_c, dc, sc_w, dw, zrows)


def _tc_paper(Ac, Aw, cc, cw, x, Wlc, Wlw, Wrc, Wrw, blc, blw):
    BM = 1000
    f32 = jnp.float32

    def body(ac_ref, aw_ref, cc_ref, cw_ref, x_ref, wlc, wlw, wrc, wrw,
             bc, bw, o_ref):
        inv_c = 1.0 / jnp.maximum(cc_ref[:, 0:1], 1.0)
        inv_w = 1.0 / jnp.maximum(cw_ref[:, 0:1], 1.0)
        p = (jnp.dot(ac_ref[0] * inv_c, wlc[0:128, :], preferred_element_type=f32)
             + jnp.dot(ac_ref[1] * inv_c, wlc[128:256, :], preferred_element_type=f32)
             + jnp.dot(aw_ref[0] * inv_w, wlw[0:128, :], preferred_element_type=f32)
             + jnp.dot(aw_ref[1] * inv_w, wlw[128:256, :], preferred_element_type=f32)
             + jnp.dot(x_ref[...], wrc[...] + wrw[...], preferred_element_type=f32)
             + bc[...] + bw[...])
        p = jnp.where(p >= 0, p, 0.01 * p)
        o_ref[...] = p.reshape(BM, 2, 128)

    grid = (N_PAPER // BM,)
    return pl.pallas_call(
        body,
        grid=grid,
        in_specs=[
            pl.BlockSpec((2, BM, 128), lambda i: (0, i, 0)),
            pl.BlockSpec((2, BM, 128), lambda i: (0, i, 0)),
            pl.BlockSpec((BM, 128), lambda i: (i, 0)),
            pl.BlockSpec((BM, 128), lambda i: (i, 0)),
            pl.BlockSpec((BM, D), lambda i: (i, 0)),
            pl.BlockSpec((D, D), lambda i: (0, 0)),
            pl.BlockSpec((D, D), lambda i: (0, 0)),
            pl.BlockSpec((D, D), lambda i: (0, 0)),
            pl.BlockSpec((D, D), lambda i: (0, 0)),
            pl.BlockSpec((1, D), lambda i: (0, 0)),
            pl.BlockSpec((1, D), lambda i: (0, 0)),
        ],
        out_specs=pl.BlockSpec((BM, 2, 128), lambda i: (i, 0, 0)),
        out_shape=jax.ShapeDtypeStruct((N_PAPER, 2, 128), f32),
    )(Ac, Aw, cc, cw, x, Wlc, Wlw, Wrc, Wrw, blc, blw)


def _tc_author(Ar, cr, x, Wlr, Wrr, blr):
    BM = 1000
    f32 = jnp.float32

    def body(ar_ref, cr_ref, x_ref, wlr, wrr, br, o_ref):
        inv_r = 1.0 / jnp.maximum(cr_ref[:, 0:1], 1.0)
        p = (jnp.dot(ar_ref[0] * inv_r, wlr[0:128, :], preferred_element_type=f32)
             + jnp.dot(ar_ref[1] * inv_r, wlr[128:256, :], preferred_element_type=f32)
             + jnp.dot(x_ref[...], wrr[...], preferred_element_type=f32)
             + br[...])
        p = jnp.where(p >= 0, p, 0.01 * p)
        o_ref[...] = p.reshape(BM, 2, 128)

    grid = (N_AUTHOR // BM,)
    return pl.pallas_call(
        body,
        grid=grid,
        in_specs=[
            pl.BlockSpec((2, BM, 128), lambda i: (0, i, 0)),
            pl.BlockSpec((BM, 128), lambda i: (i, 0)),
            pl.BlockSpec((BM, D), lambda i: (i, 0)),
            pl.BlockSpec((D, D), lambda i: (0, 0)),
            pl.BlockSpec((D, D), lambda i: (0, 0)),
            pl.BlockSpec((1, D), lambda i: (0, 0)),
        ],
        out_specs=pl.BlockSpec((BM, 2, 128), lambda i: (i, 0, 0)),
        out_shape=jax.ShapeDtypeStruct((N_AUTHOR, 2, 128), f32),
    )(Ar, cr, x, Wlr, Wrr, blr)


def _tc_final(Ac, Aw, cc, cw, xp, Wlc, Wlw, Wrc, Wrw, blc, blw, Wlin, blin):
    BM = 1000
    f32 = jnp.float32

    def body(ac_ref, aw_ref, cc_ref, cw_ref, x_ref, wlc, wlw, wrc, wrw,
             bc, bw, wl, bl, o_ref):
        inv_c = 1.0 / jnp.maximum(cc_ref[:, 0:1], 1.0)
        inv_w = 1.0 / jnp.maximum(cw_ref[:, 0:1], 1.0)
        x = x_ref[...].reshape(BM, D)
        p = (jnp.dot(ac_ref[0] * inv_c, wlc[0:128, :], preferred_element_type=f32)
             + jnp.dot(ac_ref[1] * inv_c, wlc[128:256, :], preferred_element_type=f32)
             + jnp.dot(aw_ref[0] * inv_w, wlw[0:128, :], preferred_element_type=f32)
             + jnp.dot(aw_ref[1] * inv_w, wlw[128:256, :], preferred_element_type=f32)
             + jnp.dot(x, wrc[...] + wrw[...], preferred_element_type=f32)
             + bc[...] + bw[...])
        p = jnp.where(p >= 0, p, 0.01 * p)
        o_ref[...] = jnp.dot(p, wl[...], preferred_element_type=f32) + bl[...]

    grid = (N_PAPER // BM,)
    return pl.pallas_call(
        body,
        grid=grid,
        in_specs=[
            pl.BlockSpec((2, BM, 128), lambda i: (0, i, 0)),
            pl.BlockSpec((2, BM, 128), lambda i: (0, i, 0)),
            pl.BlockSpec((BM, 128), lambda i: (i, 0)),
            pl.BlockSpec((BM, 128), lambda i: (i, 0)),
            pl.BlockSpec((BM, 2, 128), lambda i: (i, 0, 0)),
            pl.BlockSpec((D, D), lambda i: (0, 0)),
            pl.BlockSpec((D, D), lambda i: (0, 0)),
            pl.BlockSpec((D, D), lambda i: (0, 0)),
            pl.BlockSpec((D, D), lambda i: (0, 0)),
            pl.BlockSpec((1, D), lambda i: (0, 0)),
            pl.BlockSpec((1, D), lambda i: (0, 0)),
            pl.BlockSpec((D, OUT), lambda i: (0, 0)),
            pl.BlockSpec((1, OUT), lambda i: (0, 0)),
        ],
        out_specs=pl.BlockSpec((BM, OUT), lambda i: (i, 0)),
        out_shape=jax.ShapeDtypeStruct((N_PAPER, OUT), f32),
    )(Ac, Aw, cc, cw, xp, Wlc, Wlw, Wrc, Wrw, blc, blw, Wlin, blin)


def kernel(x_paper, x_author, ei_cites, ei_writes, ei_rev,
           Wl_c1, bl_c1, Wr_c1, Wl_w1, bl_w1, Wr_w1, Wl_r1, bl_r1, Wr_r1,
           Wl_c2, bl_c2, Wr_c2, Wl_w2, bl_w2, Wr_w2, Wl_r2, bl_r2, Wr_r2,
           W_lin, b_lin):
    f32 = jnp.float32
    sc_c, dc = _prep_edges(ei_cites, N_PAPER)
    sc_w, dw = _prep_edges(ei_writes, N_PAPER)
    sc_r, dr = _prep_edges(ei_rev, N_AUTHOR)
    zrows = jnp.zeros((ZR, 128), f32)
    ones128 = jnp.ones((CH, 128), f32)

    xpr = x_paper.reshape(2 * N_PAPER, 128)
    xar = x_author.reshape(2 * N_AUTHOR, 128)
    cc, cw, cr = _sc_counts(dc, dw, dr, zrows, ones128)
    Ac, Aw, Ar = _sc_layer1(xpr, xar, sc_c, dc, sc_w, dw, sc_r, dr, zrows)

    xp1 = _tc_paper(Ac.reshape(2, N_PAPER, 128), Aw.reshape(2, N_PAPER, 128),
                    cc, cw, x_paper, Wl_c1, Wl_w1, Wr_c1, Wr_w1,
                    bl_c1.reshape(1, D), bl_w1.reshape(1, D))
    xa1 = _tc_author(Ar.reshape(2, N_AUTHOR, 128), cr, x_author,
                     Wl_r1, Wr_r1, bl_r1.reshape(1, D))

    Ac2, Aw2 = _sc_layer2(xp1.reshape(2 * N_PAPER, 128),
                          xa1.reshape(2 * N_AUTHOR, 128),
                          sc_c, dc, sc_w, dw, zrows)

    return _tc_final(Ac2.reshape(2, N_PAPER, 128),
                     Aw2.reshape(2, N_PAPER, 128),
                     cc, cw, xp1, Wl_c2, Wl_w2, Wr_c2, Wr_w2,
                     bl_c2.reshape(1, D), bl_w2.reshape(1, D),
                     W_lin, b_lin.reshape(1, OUT))


# speedup vs baseline: 2.3096x; 1.0156x over previous
import functools

import jax
import jax.numpy as jnp
from jax import lax
from jax.experimental import pallas as pl
from jax.experimental.pallas import tpu as pltpu
from jax.experimental.pallas import tpu_sc as plsc

N_PAPER = 10000
N_AUTHOR = 5000
D = 256
OUT = 64
E = 160000
CH = 128
NCH = 80
IH = 40
EP = 16 * NCH * CH
ACC_ROWS = 10112
ZR = 632


def _prep_edges(ei, n_dst):
    src = ei[0].astype(jnp.int32)
    dst = ei[1].astype(jnp.int32)
    pad = EP - E
    srcp = jnp.concatenate([src, jnp.zeros((pad,), jnp.int32)])
    dstp = jnp.concatenate([dst, jnp.full((pad,), n_dst, jnp.int32)])
    src4 = jnp.stack([2 * srcp, 2 * srcp + 1]).reshape(64, IH, CH)
    dst3 = dstp.reshape(32, IH, CH)
    return src4, dst3


def _sc_phase(table, src4, dst3, isrc, idst, st0, st1, acc,
              gs0, gs1, ss0, ss1, c, s):
    w = c * 16 + s
    sts = (st0, st1)
    gss = (gs0, gs1)
    sss = (ss0, ss1)
    for h in range(2):
        pltpu.sync_copy(src4.at[w * 2 + h], isrc)
        pltpu.sync_copy(dst3.at[s * 2 + h], idst)
        pltpu.async_copy(table.at[isrc.at[0]], st0, gs0)
        pltpu.async_copy(table.at[isrc.at[1]], st1, gs1)

        def body(i, carry):
            for b in range(2):
                j = 2 * i + b
                pltpu.make_async_copy(table.at[isrc.at[j]], sts[b],
                                      gss[b]).wait()
                pltpu.async_copy(sts[b], acc.at[idst.at[j]], sss[b],
                                 add=True)
            for b in range(2):
                j = 2 * i + b
                pltpu.make_async_copy(sts[b], acc.at[idst.at[j]],
                                      sss[b]).wait()

                @pl.when(i < IH // 2 - 1)
                def _():
                    pltpu.async_copy(table.at[isrc.at[j + 2]], sts[b],
                                     gss[b])
            return carry

        lax.fori_loop(0, IH // 2, body, 0)


def _dump_papers(acc, out, c, s):
    @pl.when(s < 14)
    def _():
        r0 = s * 624
        pltpu.sync_copy(acc.at[pl.ds(r0, 624)],
                        out.at[pl.ds(c * N_PAPER + r0, 624)])

    @pl.when(s >= 14)
    def _():
        r0 = 8736 + (s - 14) * 632
        pltpu.sync_copy(acc.at[pl.ds(r0, 632)],
                        out.at[pl.ds(c * N_PAPER + r0, 632)])


def _dump_authors(acc, out, c, s):
    @pl.when(s < 15)
    def _():
        r0 = s * 312
        pltpu.sync_copy(acc.at[pl.ds(r0, 312)],
                        out.at[pl.ds(c * N_AUTHOR + r0, 312)])

    @pl.when(s >= 15)
    def _():
        pltpu.sync_copy(acc.at[pl.ds(4680, 320)],
                        out.at[pl.ds(c * N_AUTHOR + 4680, 320)])


def _sc_counts(dc, dw, dr, zrows, ones128):
    mesh = plsc.VectorSubcoreMesh(core_axis_name="c", subcore_axis_name="s")
    f32 = jnp.float32
    out_type = [
        jax.ShapeDtypeStruct((N_PAPER, 128), f32),
        jax.ShapeDtypeStruct((N_PAPER, 128), f32),
        jax.ShapeDtypeStruct((N_AUTHOR, 128), f32),
        jax.ShapeDtypeStruct((N_AUTHOR, 128), f32),
    ]
    scratch = [
        pltpu.VMEM((IH, CH), jnp.int32),
        pltpu.VMEM((CH, 128), f32),
        pltpu.VMEM_SHARED((ACC_ROWS, 128), f32),
        pltpu.SemaphoreType.DMA,
    ]

    @functools.partial(pl.kernel, mesh=mesh, out_type=out_type,
                       scratch_types=scratch)
    def k(dc_h, dw_h, dr_h, z_h, o_h, cc_o, cw_o, cr0_o, cr1_o,
          idst, stage, acc, sem):
        c = lax.axis_index("c")
        s = lax.axis_index("s")
        pltpu.sync_copy(o_h, stage)

        def count_rel(d3, halves, out, dump):
            pltpu.sync_copy(z_h, acc.at[pl.ds(s * ZR, ZR)])
            plsc.subcore_barrier()
            for h in halves:
                pltpu.sync_copy(d3.at[s * 2 + h], idst)

                def body(i, carry):
                    for b in range(4):
                        pltpu.async_copy(stage, acc.at[idst.at[4 * i + b]],
                                         sem, add=True)
                    for b in range(4):
                        pltpu.make_async_copy(stage,
                                              acc.at[idst.at[4 * i + b]],
                                              sem).wait()
                    return carry

                lax.fori_loop(0, IH // 4, body, 0)
            plsc.subcore_barrier()
            dump(acc, out, 0, s)
            plsc.subcore_barrier()

        @pl.when(c == 0)
        def _():
            count_rel(dc_h, (0, 1), cc_o, _dump_papers)
            count_rel(dr_h, (0,), cr0_o, _dump_authors)

        @pl.when(c == 1)
        def _():
            count_rel(dw_h, (0, 1), cw_o, _dump_papers)
            count_rel(dr_h, (1,), cr1_o, _dump_authors)

    return k(dc, dw, dr, zrows, ones128)


def _sc_layer1(xpr, xar, sc_c, dc, sc_w, dw, sc_r, dr, zrows):
    mesh = plsc.VectorSubcoreMesh(core_axis_name="c", subcore_axis_name="s")
    f32 = jnp.float32
    out_type = [
        jax.ShapeDtypeStruct((2 * N_PAPER, 128), f32),
        jax.ShapeDtypeStruct((2 * N_PAPER, 128), f32),
        jax.ShapeDtypeStruct((2 * N_AUTHOR, 128), f32),
    ]
    scratch = [
        pltpu.VMEM((IH, CH), jnp.int32),
        pltpu.VMEM((IH, CH), jnp.int32),
        pltpu.VMEM((CH, 128), f32),
        pltpu.VMEM((CH, 128), f32),
        pltpu.VMEM_SHARED((ACC_ROWS, 128), f32),
        pltpu.SemaphoreType.DMA,
        pltpu.SemaphoreType.DMA,
        pltpu.SemaphoreType.DMA,
        pltpu.SemaphoreType.DMA,
    ]

    @functools.partial(pl.kernel, mesh=mesh, out_type=out_type,
                       scratch_types=scratch)
    def k(xpr_h, xar_h, sc_c_h, dc_h, sc_w_h, dw_h, sc_r_h, dr_h, z_h,
          ac_o, aw_o, ar_o, isrc, idst, st0, st1, acc, gs0, gs1, ss0, ss1):
        c = lax.axis_index("c")
        s = lax.axis_index("s")
        rels = [(xpr_h, sc_c_h, dc_h, ac_o, _dump_papers),
                (xar_h, sc_w_h, dw_h, aw_o, _dump_papers),
                (xpr_h, sc_r_h, dr_h, ar_o, _dump_authors)]
        for tab, s4, d3, a_o, dump in rels:
            pltpu.sync_copy(z_h, acc.at[pl.ds(s * ZR, ZR)])
            plsc.subcore_barrier()
            _sc_phase(tab, s4, d3, isrc, idst, st0, st1, acc,
                      gs0, gs1, ss0, ss1, c, s)
            plsc.subcore_barrier()
            dump(acc, a_o, c, s)
            plsc.subcore_barrier()

    return k(xpr, xar, sc_c, dc, sc_w, dw, sc_r, dr, zrows)


def _sc_layer2(xpr, xar, sc_c, dc, sc_w, dw, zrows):
    mesh = plsc.VectorSubcoreMesh(core_axis_name="c", subcore_axis_name="s")
    f32 = jnp.float32
    out_type = [
        jax.ShapeDtypeStruct((2 * N_PAPER, 128), f32),
        jax.ShapeDtypeStruct((2 * N_PAPER, 128), f32),
    ]
    scratch = [
        pltpu.VMEM((IH, CH), jnp.int32),
        pltpu.VMEM((IH, CH), jnp.int32),
        pltpu.VMEM((CH, 128), f32),
        pltpu.VMEM((CH, 128), f32),
        pltpu.VMEM_SHARED((ACC_ROWS, 128), f32),
        pltpu.SemaphoreType.DMA,
        pltpu.SemaphoreType.DMA,
        pltpu.SemaphoreType.DMA,
        pltpu.SemaphoreType.DMA,
    ]

    @functools.partial(pl.kernel, mesh=mesh, out_type=out_type,
                       scratch_types=scratch)
    def k(xpr_h, xar_h, sc_c_h, dc_h, sc_w_h, dw_h, z_h,
          ac_o, aw_o, isrc, idst, st0, st1, acc, gs0, gs1, ss0, ss1):
        c = lax.axis_index("c")
        s = lax.axis_index("s")
        for tab, s4, d3, a_o in [(xpr_h, sc_c_h, dc_h, ac_o),
                                 (xar_h, sc_w_h, dw_h, aw_o)]:
            pltpu.sync_copy(z_h, acc.at[pl.ds(s * ZR, ZR)])
            plsc.subcore_barrier()
            _sc_phase(tab, s4, d3, isrc, idst, st0, st1, acc,
                      gs0, gs1, ss0, ss1, c, s)
            plsc.subcore_barrier()
            _dump_papers(acc, a_o, c, s)
            plsc.subcore_barrier()

    return k(xpr, xar, sc_c, dc, sc_w, dw, zrows)


def _tc_paper(Ac, Aw, cc, cw, x, Wlc, Wlw, Wrc, Wrw, blc, blw):
    BM = 1000
    f32 = jnp.float32

    def body(ac_ref, aw_ref, cc_ref, cw_ref, x_ref, wlc, wlw, wrc, wrw,
             bc, bw, o_ref):
        inv_c = 1.0 / jnp.maximum(cc_ref[:, 0:1], 1.0)
        inv_w = 1.0 / jnp.maximum(cw_ref[:, 0:1], 1.0)
        p = (jnp.dot(ac_ref[0] * inv_c, wlc[0:128, :], preferred_element_type=f32)
             + jnp.dot(ac_ref[1] * inv_c, wlc[128:256, :], preferred_element_type=f32)
             + jnp.dot(aw_ref[0] * inv_w, wlw[0:128, :], preferred_element_type=f32)
             + jnp.dot(aw_ref[1] * inv_w, wlw[128:256, :], preferred_element_type=f32)
             + jnp.dot(x_ref[...], wrc[...] + wrw[...], preferred_element_type=f32)
             + bc[...] + bw[...])
        p = jnp.where(p >= 0, p, 0.01 * p)
        o_ref[...] = p.reshape(BM, 2, 128)

    grid = (N_PAPER // BM,)
    return pl.pallas_call(
        body,
        grid=grid,
        in_specs=[
            pl.BlockSpec((2, BM, 128), lambda i: (0, i, 0)),
            pl.BlockSpec((2, BM, 128), lambda i: (0, i, 0)),
            pl.BlockSpec((BM, 128), lambda i: (i, 0)),
            pl.BlockSpec((BM, 128), lambda i: (i, 0)),
            pl.BlockSpec((BM, D), lambda i: (i, 0)),
            pl.BlockSpec((D, D), lambda i: (0, 0)),
            pl.BlockSpec((D, D), lambda i: (0, 0)),
            pl.BlockSpec((D, D), lambda i: (0, 0)),
            pl.BlockSpec((D, D), lambda i: (0, 0)),
            pl.BlockSpec((1, D), lambda i: (0, 0)),
            pl.BlockSpec((1, D), lambda i: (0, 0)),
        ],
        out_specs=pl.BlockSpec((BM, 2, 128), lambda i: (i, 0, 0)),
        out_shape=jax.ShapeDtypeStruct((N_PAPER, 2, 128), f32),
    )(Ac, Aw, cc, cw, x, Wlc, Wlw, Wrc, Wrw, blc, blw)


def _tc_author(Ar, cr0, cr1, x, Wlr, Wrr, blr):
    BM = 1000
    f32 = jnp.float32

    def body(ar_ref, cr0_ref, cr1_ref, x_ref, wlr, wrr, br, o_ref):
        inv_r = 1.0 / jnp.maximum(cr0_ref[:, 0:1] + cr1_ref[:, 0:1], 1.0)
        p = (jnp.dot(ar_ref[0] * inv_r, wlr[0:128, :], preferred_element_type=f32)
             + jnp.dot(ar_ref[1] * inv_r, wlr[128:256, :], preferred_element_type=f32)
             + jnp.dot(x_ref[...], wrr[...], preferred_element_type=f32)
             + br[...])
        p = jnp.where(p >= 0, p, 0.01 * p)
        o_ref[...] = p.reshape(BM, 2, 128)

    grid = (N_AUTHOR // BM,)
    return pl.pallas_call(
        body,
        grid=grid,
        in_specs=[
            pl.BlockSpec((2, BM, 128), lambda i: (0, i, 0)),
            pl.BlockSpec((BM, 128), lambda i: (i, 0)),
            pl.BlockSpec((BM, 128), lambda i: (i, 0)),
            pl.BlockSpec((BM, D), lambda i: (i, 0)),
            pl.BlockSpec((D, D), lambda i: (0, 0)),
            pl.BlockSpec((D, D), lambda i: (0, 0)),
            pl.BlockSpec((1, D), lambda i: (0, 0)),
        ],
        out_specs=pl.BlockSpec((BM, 2, 128), lambda i: (i, 0, 0)),
        out_shape=jax.ShapeDtypeStruct((N_AUTHOR, 2, 128), f32),
    )(Ar, cr0, cr1, x, Wlr, Wrr, blr)


def _tc_final(Ac, Aw, cc, cw, xp, Wlc, Wlw, Wrc, Wrw, blc, blw, Wlin, blin):
    BM = 1000
    f32 = jnp.float32

    def body(ac_ref, aw_ref, cc_ref, cw_ref, x_ref, wlc, wlw, wrc, wrw,
             bc, bw, wl, bl, o_ref):
        inv_c = 1.0 / jnp.maximum(cc_ref[:, 0:1], 1.0)
        inv_w = 1.0 / jnp.maximum(cw_ref[:, 0:1], 1.0)
        x = x_ref[...].reshape(BM, D)
        p = (jnp.dot(ac_ref[0] * inv_c, wlc[0:128, :], preferred_element_type=f32)
             + jnp.dot(ac_ref[1] * inv_c, wlc[128:256, :], preferred_element_type=f32)
             + jnp.dot(aw_ref[0] * inv_w, wlw[0:128, :], preferred_element_type=f32)
             + jnp.dot(aw_ref[1] * inv_w, wlw[128:256, :], preferred_element_type=f32)
             + jnp.dot(x, wrc[...] + wrw[...], preferred_element_type=f32)
             + bc[...] + bw[...])
        p = jnp.where(p >= 0, p, 0.01 * p)
        o_ref[...] = jnp.dot(p, wl[...], preferred_element_type=f32) + bl[...]

    grid = (N_PAPER // BM,)
    return pl.pallas_call(
        body,
        grid=grid,
        in_specs=[
            pl.BlockSpec((2, BM, 128), lambda i: (0, i, 0)),
            pl.BlockSpec((2, BM, 128), lambda i: (0, i, 0)),
            pl.BlockSpec((BM, 128), lambda i: (i, 0)),
            pl.BlockSpec((BM, 128), lambda i: (i, 0)),
            pl.BlockSpec((BM, 2, 128), lambda i: (i, 0, 0)),
            pl.BlockSpec((D, D), lambda i: (0, 0)),
            pl.BlockSpec((D, D), lambda i: (0, 0)),
            pl.BlockSpec((D, D), lambda i: (0, 0)),
            pl.BlockSpec((D, D), lambda i: (0, 0)),
            pl.BlockSpec((1, D), lambda i: (0, 0)),
            pl.BlockSpec((1, D), lambda i: (0, 0)),
            pl.BlockSpec((D, OUT), lambda i: (0, 0)),
            pl.BlockSpec((1, OUT), lambda i: (0, 0)),
        ],
        out_specs=pl.BlockSpec((BM, OUT), lambda i: (i, 0)),
        out_shape=jax.ShapeDtypeStruct((N_PAPER, OUT), f32),
    )(Ac, Aw, cc, cw, xp, Wlc, Wlw, Wrc, Wrw, blc, blw, Wlin, blin)


def kernel(x_paper, x_author, ei_cites, ei_writes, ei_rev,
           Wl_c1, bl_c1, Wr_c1, Wl_w1, bl_w1, Wr_w1, Wl_r1, bl_r1, Wr_r1,
           Wl_c2, bl_c2, Wr_c2, Wl_w2, bl_w2, Wr_w2, Wl_r2, bl_r2, Wr_r2,
           W_lin, b_lin):
    f32 = jnp.float32
    sc_c, dc = _prep_edges(ei_cites, N_PAPER)
    sc_w, dw = _prep_edges(ei_writes, N_PAPER)
    sc_r, dr = _prep_edges(ei_rev, N_AUTHOR)
    zrows = jnp.zeros((ZR, 128), f32)
    ones128 = jnp.ones((CH, 128), f32)

    xpr = x_paper.reshape(2 * N_PAPER, 128)
    xar = x_author.reshape(2 * N_AUTHOR, 128)
    cc, cw, cr0, cr1 = _sc_counts(dc, dw, dr, zrows, ones128)
    Ac, Aw, Ar = _sc_layer1(xpr, xar, sc_c, dc, sc_w, dw, sc_r, dr, zrows)

    xp1 = _tc_paper(Ac.reshape(2, N_PAPER, 128), Aw.reshape(2, N_PAPER, 128),
                    cc, cw, x_paper, Wl_c1, Wl_w1, Wr_c1, Wr_w1,
                    bl_c1.reshape(1, D), bl_w1.reshape(1, D))
    xa1 = _tc_author(Ar.reshape(2, N_AUTHOR, 128), cr0, cr1, x_author,
                     Wl_r1, Wr_r1, bl_r1.reshape(1, D))

    Ac2, Aw2 = _sc_layer2(xp1.reshape(2 * N_PAPER, 128),
                          xa1.reshape(2 * N_AUTHOR, 128),
                          sc_c, dc, sc_w, dw, zrows)

    return _tc_final(Ac2.reshape(2, N_PAPER, 128),
                     Aw2.reshape(2, N_PAPER, 128),
                     cc, cw, xp1, Wl_c2, Wl_w2, Wr_c2, Wr_w2,
                     bl_c2.reshape(1, D), bl_w2.reshape(1, D),
                     W_lin, b_lin.reshape(1, OUT))
